# baseline (speedup 1.0000x reference)
import functools
import math

import jax
import jax.numpy as jnp
from jax import lax
from jax.experimental import pallas as pl
from jax.experimental.pallas import tpu as pltpu
from jax.experimental.pallas import tpu_sc as plsc

D = 32
BB, LL = 4096, 200
NC, NS = 2, 16
NW = NC * NS
XROWS_PER_W = BB // NW
XROWS_PER_CHUNK = 8
CHUNK = XROWS_PER_CHUNK * LL
NCHUNK = XROWS_PER_W // XROWS_PER_CHUNK
SCALE = math.sqrt(D)

_mesh = plsc.VectorSubcoreMesh(
    core_axis_name="c", subcore_axis_name="s", num_cores=NC, num_subcores=NS
)


@functools.partial(
    pl.kernel,
    out_type=jax.ShapeDtypeStruct((BB, LL, D), jnp.float32),
    mesh=_mesh,
    scratch_types=[
        pltpu.VMEM((CHUNK,), jnp.int32),
        pltpu.VMEM((CHUNK,), jnp.int32),
        pltpu.VMEM((CHUNK, D), jnp.float32),
        pltpu.VMEM((CHUNK, D), jnp.float32),
        pltpu.SemaphoreType.DMA,
        pltpu.SemaphoreType.DMA,
        pltpu.SemaphoreType.DMA,
        pltpu.SemaphoreType.DMA,
    ],
    compiler_params=pltpu.CompilerParams(use_tc_tiling_on_sc=False),
)
def _embed_lookup(idx_hbm, table_hbm, out_hbm, idx0, idx1, rows0, rows1,
                  gsem0, gsem1, osem0, osem1):
    wid = lax.axis_index("s") * NC + lax.axis_index("c")
    base = wid * XROWS_PER_W

    def load_idx(c, idx_v):
        for a in range(XROWS_PER_CHUNK):
            pltpu.sync_copy(
                idx_hbm.at[base + c * XROWS_PER_CHUNK + a],
                idx_v.at[pl.ds(a * LL, LL)],
            )

    def start_gather(idx_v, rows_v, sem):
        return pltpu.async_copy(table_hbm.at[idx_v], rows_v, sem)

    def scale(rows_v):
        @plsc.parallel_loop(0, CHUNK, 1, unroll=8)
        def _(i):
            for j in range(2):
                sl = pl.ds(j * 16, 16)
                rows_v[i, sl] = rows_v[i, sl] * SCALE

    def start_out(c, rows_v, sem):
        for a in range(XROWS_PER_CHUNK):
            pltpu.async_copy(
                rows_v.at[pl.ds(a * LL, LL)],
                out_hbm.at[base + c * XROWS_PER_CHUNK + a],
                sem,
            )

    def wait_gather(idx_v, rows_v, sem):
        pltpu.make_async_copy(table_hbm.at[idx_v], rows_v, sem).wait()

    def wait_out(c, rows_v, sem):
        for a in range(XROWS_PER_CHUNK):
            pltpu.make_async_copy(
                rows_v.at[pl.ds(a * LL, LL)],
                out_hbm.at[base + c * XROWS_PER_CHUNK + a],
                sem,
            ).wait()

    load_idx(0, idx0)
    start_gather(idx0, rows0, gsem0)

    wait_gather(idx0, rows0, gsem0)
    load_idx(1, idx1)
    start_gather(idx1, rows1, gsem1)
    scale(rows0)
    start_out(0, rows0, osem0)

    def pair_body(s, _):
        c1 = 2 * s + 1
        c2 = 2 * s + 2
        wait_gather(idx1, rows1, gsem1)
        load_idx(c1 + 1, idx0)
        wait_out(c1 - 1, rows0, osem0)
        start_gather(idx0, rows0, gsem0)
        scale(rows1)
        start_out(c1, rows1, osem1)
        wait_gather(idx0, rows0, gsem0)
        load_idx(c2 + 1, idx1)
        wait_out(c2 - 1, rows1, osem1)
        start_gather(idx1, rows1, gsem1)
        scale(rows0)
        start_out(c2, rows0, osem0)
        return 0

    lax.fori_loop(0, (NCHUNK - 2) // 2, pair_body, 0)

    wait_gather(idx1, rows1, gsem1)
    scale(rows1)
    start_out(NCHUNK - 1, rows1, osem1)
    wait_out(NCHUNK - 2, rows0, osem0)
    wait_out(NCHUNK - 1, rows1, osem1)


def kernel(x, table):
    return _embed_lookup(x, table)


# speedup vs baseline: 1.7251x; 1.7251x over previous
import functools
import math

import jax
import jax.numpy as jnp
from jax import lax
from jax.experimental import pallas as pl
from jax.experimental.pallas import tpu as pltpu
from jax.experimental.pallas import tpu_sc as plsc

D = 32
BB, LL = 4096, 200
NC, NS = 2, 16
NW = NC * NS
BT = BB // 128
CL = 4
CHUNK = CL * 128
NCH = LL // CL
PAD = 129
SCALE = math.sqrt(D)

_mesh = plsc.VectorSubcoreMesh(
    core_axis_name="c", subcore_axis_name="s", num_cores=NC, num_subcores=NS
)


@functools.partial(
    pl.kernel,
    out_type=jax.ShapeDtypeStruct((LL, 4, BT, 8, 128), jnp.float32),
    mesh=_mesh,
    scratch_types=[
        pltpu.VMEM((128, LL), jnp.int32),
        pltpu.VMEM((CHUNK,), jnp.int32),
        pltpu.VMEM((CHUNK,), jnp.int32),
        pltpu.VMEM((CHUNK,), jnp.int32),
        pltpu.VMEM((CHUNK, D), jnp.float32),
        pltpu.VMEM((CHUNK, D), jnp.float32),
        pltpu.VMEM((CHUNK, D), jnp.float32),
        pltpu.VMEM((CL, 4, 8, PAD), jnp.float32),
        pltpu.VMEM((CL, 4, 8, PAD), jnp.float32),
        pltpu.SemaphoreType.DMA,
        pltpu.SemaphoreType.DMA,
        pltpu.SemaphoreType.DMA,
        pltpu.SemaphoreType.DMA,
        pltpu.SemaphoreType.DMA,
    ],
    compiler_params=pltpu.CompilerParams(
        use_tc_tiling_on_sc=False, needs_layout_passes=False
    ),
)
def _embed_lookup(x_hbm, table_hbm, out_hbm,
                  xblk, idx0, idx1, idx2, rows0, rows1, rows2, t0, t1,
                  g0, g1, g2, o0, o1):
    wid = lax.axis_index("s") * NC + lax.axis_index("c")
    idxs = (idx0, idx1, idx2)
    rows = (rows0, rows1, rows2)
    gsem = (g0, g1, g2)
    ts = (t0, t1)
    osem = (o0, o1)

    iota = lax.iota(jnp.int32, 16)
    e0v = iota & 7
    gv0 = iota >> 3
    gv1 = gv0 + 2

    def build_idx(c, k):
        for lc in range(CL):
            l = c * CL + lc
            colv = jnp.broadcast_to(l, (16,)).astype(jnp.int32)
            for bs in range(8):
                rv = plsc.load_gather(xblk, [bs * 16 + iota, colv])
                idxs[k][pl.ds(lc * 128 + bs * 16, 16)] = rv

    def start_gather(k):
        pltpu.async_copy(table_hbm.at[idxs[k]], rows[k], gsem[k])

    def wait_gather(k):
        pltpu.make_async_copy(table_hbm.at[idxs[k]], rows[k], gsem[k]).wait()

    def transpose(k, tk):
        @plsc.parallel_loop(0, CHUNK, 1, unroll=4)
        def _(i):
            lc = i >> 7
            bb = i & 127
            lcv = jnp.broadcast_to(lc, (16,))
            bv = jnp.broadcast_to(bb, (16,))
            v0 = rows[k][i, pl.ds(0, 16)] * SCALE
            v1 = rows[k][i, pl.ds(16, 16)] * SCALE
            plsc.store_scatter(ts[tk], [lcv, gv0, e0v, bv], v0)
            plsc.store_scatter(ts[tk], [lcv, gv1, e0v, bv], v1)

    def start_out(c, tk):
        for lc in range(CL):
            l = c * CL + lc
            for g in range(4):
                pltpu.async_copy(
                    ts[tk].at[lc, g, :, pl.ds(0, 128)],
                    out_hbm.at[l, g, wid],
                    osem[tk],
                )

    def wait_out(c, tk):
        for lc in range(CL):
            l = c * CL + lc
            for g in range(4):
                pltpu.make_async_copy(
                    ts[tk].at[lc, g, :, pl.ds(0, 128)],
                    out_hbm.at[l, g, wid],
                    osem[tk],
                ).wait()

    def chunk_body(c, rb, with_build=True, with_outwait=True):
        nb = (rb + 2) % 3
        tb = c % 2 if isinstance(c, int) else None
        wait_gather(rb)
        if with_build:
            build_idx(c + 2, nb)
            start_gather(nb)
        if with_outwait:
            wait_out(c - 2, tb)
        transpose(rb, tb)
        start_out(c, tb)

    pltpu.sync_copy(x_hbm.at[pl.ds(wid * 128, 128)], xblk)

    build_idx(0, 0)
    start_gather(0)
    build_idx(1, 1)
    start_gather(1)

    chunk_body(0, 0, with_outwait=False)
    chunk_body(1, 1, with_outwait=False)

    def group(s, _):
        for k in range(6):
            c = 2 + s * 6 + k
            rb = (2 + k) % 3
            nb = (rb + 2) % 3
            tb = k % 2
            wait_gather(rb)
            build_idx(c + 2, nb)
            start_gather(nb)
            wait_out(c - 2, tb)
            transpose(rb, tb)
            start_out(c, tb)
        return 0

    lax.fori_loop(0, 7, group, 0)

    chunk_body(44, 2)
    chunk_body(45, 0)
    chunk_body(46, 1)
    chunk_body(47, 2)

    chunk_body(48, 0, with_build=False)
    chunk_body(49, 1, with_build=False)

    wait_out(48, 0)
    wait_out(49, 1)


def kernel(x, table):
    a = _embed_lookup(x, table)
    return a.transpose(2, 4, 0, 1, 3).reshape(BB, LL, D)
